# baseline (speedup 1.0000x reference)
import jax
import jax.numpy as jnp
from jax import lax
from jax.experimental import pallas as pl

_H = 128
_W = 128
_P = 32
_RB = _H * _W * _P // 128


def _raster_block(scal_ref, xs_ref, ys_ref, out_ref):
    rx = scal_ref[0, 0, 0]
    ry = scal_ref[0, 0, 1]
    ox = scal_ref[0, 0, 2]
    oy = scal_ref[0, 0, 3]
    xs = xs_ref[0]
    ys = ys_ref[0]
    col = (xs / rx + ox).astype(jnp.int32)
    row = (ys / ry + oy).astype(jnp.int32)
    ri = lax.broadcasted_iota(jnp.int32, (_RB, 128), 0)
    li = lax.broadcasted_iota(jnp.int32, (_RB, 128), 1)
    h = ri >> 5
    w = ((ri & 31) << 2) + (li >> 5)
    hit = (h == row) & (w == col)
    out_ref[...] = hit.astype(jnp.float32)


def kernel(x, resolution, origin):
    B = x.shape[0]
    pts = x.reshape(B, _P, 2)
    xs = jnp.tile(pts[:, :, 0], (1, 128 // _P)).reshape(B, 1, 128)
    ys = jnp.tile(pts[:, :, 1], (1, 128 // _P)).reshape(B, 1, 128)
    scal = jnp.concatenate(
        [resolution, origin, jnp.zeros((B, 4), jnp.float32)], axis=1
    ).reshape(B, 1, 8)
    out = pl.pallas_call(
        _raster_block,
        grid=(B,),
        in_specs=[
            pl.BlockSpec((1, 1, 8), lambda b: (b, 0, 0)),
            pl.BlockSpec((1, 1, 128), lambda b: (b, 0, 0)),
            pl.BlockSpec((1, 1, 128), lambda b: (b, 0, 0)),
        ],
        out_specs=pl.BlockSpec((_RB, 128), lambda b: (b, 0)),
        out_shape=jax.ShapeDtypeStruct((B * _RB, 128), jnp.float32),
    )(scal, xs, ys)
    return out.reshape(B, _H, _W, _P)


# speedup vs baseline: 8.3744x; 8.3744x over previous
import jax
import jax.numpy as jnp
from jax import lax
from jax.experimental import pallas as pl

_H = 128
_W = 128
_P = 32
_BH = 128


def _raster_block(scal_ref, xs_ref, ys_ref, out_ref):
    rx = scal_ref[0, 0, 0]
    ry = scal_ref[0, 0, 1]
    ox = scal_ref[0, 0, 2]
    oy = scal_ref[0, 0, 3]
    col = (xs_ref[...] / rx + ox).astype(jnp.int32)
    row = (ys_ref[...] / ry + oy).astype(jnp.int32)
    hblk = pl.program_id(1)
    ih = lax.broadcasted_iota(jnp.int32, (1, _BH, 1, 1), 1) + hblk * _BH
    iw = lax.broadcasted_iota(jnp.int32, (1, 1, _P, _W), 3)
    hit = (ih == row) & (iw == col)
    out_ref[...] = hit.astype(jnp.float32)


def kernel(x, resolution, origin):
    B = x.shape[0]
    pts = x.reshape(B, _P, 2)
    xs = jnp.broadcast_to(pts[:, :, 0][:, None, :, None], (B, 1, _P, _W))
    ys = jnp.broadcast_to(pts[:, :, 1][:, None, :, None], (B, 1, _P, _W))
    scal = jnp.concatenate(
        [resolution, origin, jnp.zeros((B, 4), jnp.float32)], axis=1
    ).reshape(B, 1, 8)
    out = pl.pallas_call(
        _raster_block,
        grid=(B, _H // _BH),
        in_specs=[
            pl.BlockSpec((1, 1, 8), lambda b, h: (b, 0, 0)),
            pl.BlockSpec((1, 1, _P, _W), lambda b, h: (b, 0, 0, 0)),
            pl.BlockSpec((1, 1, _P, _W), lambda b, h: (b, 0, 0, 0)),
        ],
        out_specs=pl.BlockSpec((1, _BH, _P, _W), lambda b, h: (b, h, 0, 0)),
        out_shape=jax.ShapeDtypeStruct((B, _H, _P, _W), jnp.float32),
    )(scal, xs, ys)
    return jnp.transpose(out, (0, 1, 3, 2))


# speedup vs baseline: 10.8771x; 1.2989x over previous
import functools

import jax
import jax.numpy as jnp
from jax import lax
from jax.experimental import pallas as pl
from jax.experimental.pallas import tpu as pltpu
from jax.experimental.pallas import tpu_sc as plsc

_B = 256
_H = 128
_W = 128
_P = 32
_NW = 32
_BPW = _B // _NW
_CH = 16
_NC = _H // _CH
_CHW = _CH * _P * _W


def _sc_body(xc_hbm, yc_hbm, scal_hbm, zin_hbm, out_hbm, xv, yv, sv, buf):
    wid = lax.axis_index("s") * 2 + lax.axis_index("c")
    pltpu.sync_copy(zin_hbm, buf)
    ones = jnp.full((16,), 1.0, jnp.float32)
    zeros16 = jnp.zeros((16,), jnp.float32)
    zi16 = jnp.zeros((16,), jnp.int32)
    iota = lax.broadcasted_iota(jnp.int32, (16,), 0)
    for bi in range(_BPW):
        b = wid * _BPW + bi
        pltpu.sync_copy(xc_hbm.at[pl.ds(b, 1)], xv)
        pltpu.sync_copy(yc_hbm.at[pl.ds(b, 1)], yv)
        pltpu.sync_copy(scal_hbm.at[pl.ds(b, 1)], sv)
        rx = sv[0, 0]
        ry = sv[0, 1]
        ox = sv[0, 2]
        oy = sv[0, 3]
        cols = []
        rows = []
        ips = []
        for g in range(2):
            cols.append((xv[0, g] / rx + ox).astype(jnp.int32))
            rows.append((yv[0, g] / ry + oy).astype(jnp.int32))
            ips.append(iota + 16 * g)
        for c in range(_NC):
            fis = []
            ms = []
            for g in range(2):
                lh = rows[g] - c * _CH
                m = (lh >= 0) & (lh < _CH)
                fi = lh * (_P * _W) + ips[g] * _W + cols[g]
                fis.append(fi)
                ms.append(m)
                plsc.store_scatter(buf, [fi], ones, mask=m)
            pltpu.sync_copy(
                buf, out_hbm.at[pl.ds(b * (_H * _P * _W) + c * _CHW, _CHW)]
            )
            for g in range(2):
                plsc.store_scatter(buf, [fis[g]], zeros16, mask=ms[g])


def kernel(x, resolution, origin):
    B = x.shape[0]
    pts = x.reshape(B, _P, 2)
    xc = pts[:, :, 0].reshape(B, 2, 16)
    yc = pts[:, :, 1].reshape(B, 2, 16)
    scal = jnp.stack(
        [
            jnp.broadcast_to(resolution[:, 0:1], (B, 16)),
            jnp.broadcast_to(resolution[:, 1:2], (B, 16)),
            jnp.broadcast_to(origin[:, 0:1], (B, 16)),
            jnp.broadcast_to(origin[:, 1:2], (B, 16)),
        ],
        axis=1,
    )
    zin = jnp.zeros((_CHW,), jnp.float32)
    run = functools.partial(
        pl.kernel,
        out_type=jax.ShapeDtypeStruct((B * _H * _P * _W,), jnp.float32),
        mesh=plsc.VectorSubcoreMesh(core_axis_name="c", subcore_axis_name="s"),
        compiler_params=pltpu.CompilerParams(
            needs_layout_passes=False, use_tc_tiling_on_sc=False
        ),
        scratch_types=[
            pltpu.VMEM((1, 2, 16), jnp.float32),
            pltpu.VMEM((1, 2, 16), jnp.float32),
            pltpu.VMEM((1, 4, 16), jnp.float32),
            pltpu.VMEM((_CHW,), jnp.float32),
        ],
    )(_sc_body)
    out = run(xc, yc, scal, zin)
    return jnp.transpose(out.reshape(B, _H, _P, _W), (0, 1, 3, 2))


# speedup vs baseline: 11.0742x; 1.0181x over previous
import functools

import jax
import jax.numpy as jnp
from jax import lax
from jax.experimental import pallas as pl
from jax.experimental.pallas import tpu as pltpu
from jax.experimental.pallas import tpu_sc as plsc

_B = 256
_H = 128
_W = 128
_P = 32
_NW = 32
_BPW = _B // _NW
_CH = 16
_NCH = _H // _CH
_CHW = _CH * _P * _W
_IMG = _H * _P * _W


def _sc_body(xc_hbm, yc_hbm, scal_hbm, zin_hbm, out_hbm, xv, yv, sv, zbuf,
             idxv, onev, fsem, ssem):
    wid = lax.axis_index("s") * 2 + lax.axis_index("c")
    pltpu.sync_copy(zin_hbm, zbuf)
    iota = lax.broadcasted_iota(jnp.int32, (16,), 0)
    ones = jnp.full((16,), 1.0, jnp.float32)
    for i in range(8):
        onev[pl.ds(i * 16, 16)] = ones
    prev = None
    for bi in range(_BPW):
        b = wid * _BPW + bi
        pltpu.sync_copy(xc_hbm.at[pl.ds(b, 1)], xv)
        pltpu.sync_copy(yc_hbm.at[pl.ds(b, 1)], yv)
        pltpu.sync_copy(scal_hbm.at[pl.ds(b, 1)], sv)
        rx = sv[0, 0]
        ry = sv[0, 1]
        ox = sv[0, 2]
        oy = sv[0, 3]
        for g in range(2):
            col = (xv[0, g] / rx + ox).astype(jnp.int32)
            row = (yv[0, g] / ry + oy).astype(jnp.int32)
            ip = iota + 16 * g
            fi = b * _IMG + row * (_P * _W) + ip * _W + col
            k = bi * 2 + g
            idxv[k // 8, pl.ds((k % 8) * 16, 16)] = fi
        handles = []
        for c in range(_NCH):
            handles.append(
                pltpu.async_copy(
                    zbuf, out_hbm.at[pl.ds(b * _IMG + c * _CHW, _CHW)], fsem
                )
            )
        if prev is not None:
            for h in prev:
                h.wait()
        prev = handles
    for h in prev:
        h.wait()
    s0 = pltpu.async_copy(onev, out_hbm.at[idxv.at[0]], ssem)
    s1 = pltpu.async_copy(onev, out_hbm.at[idxv.at[1]], ssem)
    s0.wait()
    s1.wait()


def kernel(x, resolution, origin):
    B = x.shape[0]
    pts = x.reshape(B, _P, 2)
    xc = pts[:, :, 0].reshape(B, 2, 16)
    yc = pts[:, :, 1].reshape(B, 2, 16)
    scal = jnp.stack(
        [
            jnp.broadcast_to(resolution[:, 0:1], (B, 16)),
            jnp.broadcast_to(resolution[:, 1:2], (B, 16)),
            jnp.broadcast_to(origin[:, 0:1], (B, 16)),
            jnp.broadcast_to(origin[:, 1:2], (B, 16)),
        ],
        axis=1,
    )
    zin = jnp.zeros((_CHW,), jnp.float32)
    run = functools.partial(
        pl.kernel,
        out_type=jax.ShapeDtypeStruct((B * _IMG,), jnp.float32),
        mesh=plsc.VectorSubcoreMesh(core_axis_name="c", subcore_axis_name="s"),
        compiler_params=pltpu.CompilerParams(
            needs_layout_passes=False, use_tc_tiling_on_sc=False
        ),
        scratch_types=[
            pltpu.VMEM((1, 2, 16), jnp.float32),
            pltpu.VMEM((1, 2, 16), jnp.float32),
            pltpu.VMEM((1, 4, 16), jnp.float32),
            pltpu.VMEM((_CHW,), jnp.float32),
            pltpu.VMEM((2, 128), jnp.int32),
            pltpu.VMEM((128,), jnp.float32),
            pltpu.SemaphoreType.DMA,
            pltpu.SemaphoreType.DMA,
        ],
    )(_sc_body)
    out = run(xc, yc, scal, zin)
    return jnp.transpose(out.reshape(B, _H, _P, _W), (0, 1, 3, 2))
